# Initial kernel scaffold; baseline (speedup 1.0000x reference)
import jax
import jax.numpy as jnp
from jax.experimental import pallas as pl


def kernel(x, edge_index, batch, W1, b1, W2, b2, W3, b3, fc_W, fc_b):
    raise NotImplementedError("write your pallas kernel here")


# speedup vs baseline: 33.3916x; 33.3916x over previous
import functools

import jax
import jax.numpy as jnp
from jax import lax
from jax.experimental import pallas as pl
from jax.experimental.pallas import tpu as pltpu
from jax.experimental.pallas import tpu_sc as plsc

NN = 100000
EE = 3200000
GG = 1024
NC = 2
NS = 16
SUB = 128
IR = 8
CHUNK = IR * SUB
EP = 3211264
BT = 512
NP = 100352
NSTEP = NP // BT
RPT = NP // NS

@functools.cache
def _sc_mesh():
    return plsc.VectorSubcoreMesh(core_axis_name="c", subcore_axis_name="s",
                                  num_cores=NC, num_subcores=NS)


def _edge_loop(src2d, dst2d, table, acc, src_v, dst_v, rows_v,
               sem_g, sem_s, row_start, nchunks):
    def chunk(k, carry):
        rb = row_start + k * IR
        pltpu.sync_copy(src2d.at[pl.ds(rb, IR)], src_v)
        pltpu.sync_copy(dst2d.at[pl.ds(rb, IR)], dst_v)
        gs = [pltpu.async_copy(table.at[src_v.at[j]], rows_v.at[j], sem_g)
              for j in range(IR)]
        for cp in gs:
            cp.wait()
        ss = [pltpu.async_copy(rows_v.at[j], acc.at[dst_v.at[j]], sem_s,
                               add=True)
              for j in range(IR)]
        for cp in ss:
            cp.wait()
        return carry
    lax.fori_loop(0, nchunks, chunk, 0)


def _stripe(s):
    return pl.ds(s * RPT, RPT)


def _deg_body(dst2d, ones_hbm, zeros_hbm, out_a, out_b, idx_v, ones_v, acc):
    c = lax.axis_index("c")
    s = lax.axis_index("s")
    pltpu.sync_copy(zeros_hbm.at[_stripe(s)], acc.at[_stripe(s)])
    pltpu.sync_copy(ones_hbm, ones_v)
    plsc.subcore_barrier()
    w = s * NC + c
    nchunks = EP // (NC * NS * CHUNK)
    def chunk(k, carry):
        rb = (w * nchunks + k) * IR
        pltpu.sync_copy(dst2d.at[pl.ds(rb, IR)], idx_v)
        for j in range(IR):
            pltpu.sync_copy(ones_v, acc.at[idx_v.at[j]], add=True)
        return carry
    lax.fori_loop(0, nchunks, chunk, 0)
    plsc.subcore_barrier()
    @pl.when(c == 0)
    def _():
        pltpu.sync_copy(acc.at[_stripe(s)], out_a.at[_stripe(s)])
    @pl.when(c == 1)
    def _():
        pltpu.sync_copy(acc.at[_stripe(s)], out_b.at[_stripe(s)])


def _agg_body(edge_split, src_a2d, src_b2d, dst2d, table8, zeros_hbm,
              out_a, out_b, src_v, dst_v, rows_v, acc, sem_g, sem_s):
    c = lax.axis_index("c")
    s = lax.axis_index("s")
    pltpu.sync_copy(zeros_hbm.at[_stripe(s)], acc.at[_stripe(s)])
    plsc.subcore_barrier()
    if edge_split:
        w = s * NC + c
        nchunks = EP // (NC * NS * CHUNK)
        _edge_loop(src_a2d, dst2d, table8, acc, src_v, dst_v, rows_v,
                   sem_g, sem_s, w * nchunks * IR, nchunks)
    else:
        nchunks = EP // (NS * CHUNK)
        row_start = s * nchunks * IR
        @pl.when(c == 0)
        def _():
            _edge_loop(src_a2d, dst2d, table8, acc, src_v, dst_v, rows_v,
                       sem_g, sem_s, row_start, nchunks)
        @pl.when(c == 1)
        def _():
            _edge_loop(src_b2d, dst2d, table8, acc, src_v, dst_v, rows_v,
                       sem_g, sem_s, row_start, nchunks)
    plsc.subcore_barrier()
    @pl.when(c == 0)
    def _():
        pltpu.sync_copy(acc.at[_stripe(s)], out_a.at[_stripe(s)])
    @pl.when(c == 1)
    def _():
        pltpu.sync_copy(acc.at[_stripe(s)], out_b.at[_stripe(s)])


@functools.cache
def _sc_kernels():
    params = pltpu.CompilerParams(use_tc_tiling_on_sc=False)
    deg = pl.kernel(
        _deg_body,
        out_type=[jax.ShapeDtypeStruct((NP,), jnp.float32)] * 2,
        mesh=_sc_mesh(),
        compiler_params=params,
        scratch_types=[
            pltpu.VMEM((IR, SUB), jnp.int32),
            pltpu.VMEM((SUB,), jnp.float32),
            pltpu.VMEM_SHARED((NP,), jnp.float32),
        ],
    )
    aggs = []
    for edge_split in (True, False):
        aggs.append(pl.kernel(
            functools.partial(_agg_body, edge_split),
            out_type=[jax.ShapeDtypeStruct((NP, 16), jnp.float32)] * 2,
            mesh=_sc_mesh(),
            compiler_params=params,
            scratch_types=[
                pltpu.VMEM((IR, SUB), jnp.int32),
                pltpu.VMEM((IR, SUB), jnp.int32),
                pltpu.VMEM((IR, SUB, 16), jnp.float32),
                pltpu.VMEM_SHARED((NP, 16), jnp.float32),
                pltpu.SemaphoreType.DMA,
                pltpu.SemaphoreType.DMA,
            ],
        ))
    return deg, aggs[0], aggs[1]


def _prep_body(deg_a, deg_b, x_ref, misc_ref):
    deg = deg_a[...] + deg_b[...] + 1.0
    dis = lax.rsqrt(deg)
    g1 = x_ref[...] * dis
    pad = jnp.zeros((BT, 30), jnp.float32)
    tail = jnp.zeros((BT, 95), jnp.float32)
    misc_ref[...] = jnp.concatenate([g1, pad, dis, tail], axis=1)


def _mid1_body(e1a, e1b, misc, w1, b1, out_ref):
    mv = misc[...]
    dis = mv[:, 32:33]
    z = (e1a[...][:, 0:2] + e1b[...][:, 0:2] + mv[:, 0:2]) * dis
    h1 = jnp.maximum(jnp.dot(z, w1[...], preferred_element_type=jnp.float32)
                     + b1[...], 0.0)
    g2 = h1 * dis
    tail = jnp.zeros((BT, 95), jnp.float32)
    out_ref[...] = jnp.concatenate([g2, dis, tail], axis=1)


def _mid2_body(e2a, e2b, g2t, w2, b2, w3, out_ref):
    gv = g2t[...]
    dis = gv[:, 32:33]
    u = (jnp.concatenate([e2a[...], e2b[...]], axis=1) + gv[:, 0:32]) * dis
    h2 = jnp.maximum(jnp.dot(u, w2[...], preferred_element_type=jnp.float32)
                     + b2[...], 0.0)
    t3 = jnp.dot(h2, w3[...], preferred_element_type=jnp.float32)
    g3 = t3 * dis
    tail = jnp.zeros((BT, 95), jnp.float32)
    out_ref[...] = jnp.concatenate([g3, dis, tail], axis=1)


def _final_body(e3a, e3b, g3t, batch, b3, fcw, fcb, out_ref, acc):
    i = pl.program_id(0)
    @pl.when(i == 0)
    def _():
        acc[...] = jnp.zeros_like(acc)
    gv = g3t[...]
    z = (jnp.concatenate([e3a[...], e3b[...]], axis=1)
         + gv[:, 0:32]) * gv[:, 32:33]
    h3 = jnp.maximum(z + b3[...], 0.0)
    y = jnp.dot(h3, fcw[...], preferred_element_type=jnp.float32)
    yo = jnp.concatenate([y, jnp.ones_like(y)], axis=1)
    onehot_t = (lax.broadcasted_iota(jnp.int32, (GG, BT), 0)
                == batch[...]).astype(jnp.float32)
    acc[...] += jnp.dot(onehot_t, yo, preferred_element_type=jnp.float32)
    @pl.when(i == NSTEP - 1)
    def _():
        a = acc[...]
        out_ref[...] = a[:, 0:1] / jnp.maximum(a[:, 1:2], 1.0) + fcb[...]


def _row_spec(width):
    return pl.BlockSpec((BT, width), lambda i: (i, 0))


def _full_spec(shape):
    return pl.BlockSpec(shape, lambda i: tuple(0 for _ in shape))


_prep_call = pl.pallas_call(
    _prep_body,
    grid=(NSTEP,),
    in_specs=[_row_spec(1), _row_spec(1), _row_spec(2)],
    out_specs=_row_spec(128),
    out_shape=jax.ShapeDtypeStruct((NP, 128), jnp.float32),
)

_mid1_call = pl.pallas_call(
    _mid1_body,
    grid=(NSTEP,),
    in_specs=[_row_spec(16), _row_spec(16), _row_spec(128),
              _full_spec((2, 32)), _full_spec((1, 32))],
    out_specs=_row_spec(128),
    out_shape=jax.ShapeDtypeStruct((NP, 128), jnp.float32),
)

_mid2_call = pl.pallas_call(
    _mid2_body,
    grid=(NSTEP,),
    in_specs=[_row_spec(16), _row_spec(16), _row_spec(128),
              _full_spec((32, 64)), _full_spec((1, 64)),
              _full_spec((64, 32))],
    out_specs=_row_spec(128),
    out_shape=jax.ShapeDtypeStruct((NP, 128), jnp.float32),
)

_final_call = pl.pallas_call(
    _final_body,
    grid=(NSTEP,),
    in_specs=[_row_spec(16), _row_spec(16), _row_spec(128),
              pl.BlockSpec((1, BT), lambda i: (0, i)),
              _full_spec((1, 32)), _full_spec((32, 1)), _full_spec((1, 1))],
    out_specs=_full_spec((GG, 1)),
    out_shape=jax.ShapeDtypeStruct((GG, 1), jnp.float32),
    scratch_shapes=[pltpu.VMEM((GG, 2), jnp.float32)],
)


def kernel(x, edge_index, batch, W1, b1, W2, b2, W3, b3, fc_W, fc_b):
    f32 = jnp.float32
    i32 = jnp.int32
    npad = EP - EE
    pad_src = (jnp.arange(npad, dtype=i32) * 7919) % NN
    pad_dst = NN + (jnp.arange(npad, dtype=i32) % (NP - NN))
    src = jnp.concatenate([edge_index[0], pad_src])
    src_a2d = (src * 8).reshape(EP // SUB, SUB)
    src_b2d = (src * 8 + 1).reshape(EP // SUB, SUB)
    dst2d = jnp.concatenate([edge_index[1], pad_dst]).reshape(EP // SUB, SUB)

    xp = jnp.zeros((NP, 2), f32).at[:NN].set(x.astype(f32))
    batchp = jnp.full((NP,), GG + 7, i32).at[:NN].set(batch).reshape(1, NP)

    ones_sub = jnp.ones((SUB,), f32)
    z1 = jnp.zeros((NP,), f32)
    z16 = jnp.zeros((NP, 16), f32)

    _deg_call, _agg_es_call, _agg_fs_call = _sc_kernels()
    deg_a, deg_b = _deg_call(dst2d, ones_sub, z1)
    misc1 = _prep_call(deg_a.reshape(NP, 1), deg_b.reshape(NP, 1), xp)
    e1a, e1b = _agg_es_call(src_a2d, src_b2d, dst2d,
                            misc1.reshape(NP * 8, 16), z16)
    g2t = _mid1_call(e1a, e1b, misc1, W1, b1.reshape(1, 32))
    e2a, e2b = _agg_fs_call(src_a2d, src_b2d, dst2d,
                            g2t.reshape(NP * 8, 16), z16)
    g3t = _mid2_call(e2a, e2b, g2t, W2, b2.reshape(1, 64), W3)
    e3a, e3b = _agg_fs_call(src_a2d, src_b2d, dst2d,
                            g3t.reshape(NP * 8, 16), z16)
    out = _final_call(e3a, e3b, g3t, batchp, b3.reshape(1, 32),
                      fc_W, fc_b.reshape(1, 1))
    return out


# speedup vs baseline: 34.9649x; 1.0471x over previous
import functools

import jax
import jax.numpy as jnp
from jax import lax
from jax.experimental import pallas as pl
from jax.experimental.pallas import tpu as pltpu
from jax.experimental.pallas import tpu_sc as plsc

NN = 100000
EE = 3200000
GG = 1024
NC = 2
NS = 16
SUB = 128
IR = 8
CHUNK = IR * SUB
EP = 3211264
BT = 512
NP = 100352
NSTEP = NP // BT
RPT = NP // NS

@functools.cache
def _sc_mesh():
    return plsc.VectorSubcoreMesh(core_axis_name="c", subcore_axis_name="s",
                                  num_cores=NC, num_subcores=NS)


def _edge_loop(src2d, dst2d, table, acc, src_v, dst_v, rows_v,
               sem_g, sem_s, row_start, nchunks):
    def chunk(k, carry):
        rb = row_start + k * IR
        pltpu.sync_copy(src2d.at[pl.ds(rb, IR)], src_v)
        pltpu.sync_copy(dst2d.at[pl.ds(rb, IR)], dst_v)
        gs = [pltpu.async_copy(table.at[src_v.at[j]], rows_v.at[j], sem_g)
              for j in range(IR)]
        for cp in gs:
            cp.wait()
        ss = [pltpu.async_copy(rows_v.at[j], acc.at[dst_v.at[j]], sem_s,
                               add=True)
              for j in range(IR)]
        for cp in ss:
            cp.wait()
        return carry
    lax.fori_loop(0, nchunks, chunk, 0)


def _stripe(s):
    return pl.ds(s * RPT, RPT)


def _deg_body(dst2d, ones_hbm, zeros_hbm, out_a, out_b, idx_v, ones_v, acc):
    c = lax.axis_index("c")
    s = lax.axis_index("s")
    pltpu.sync_copy(zeros_hbm.at[_stripe(s)], acc.at[_stripe(s)])
    pltpu.sync_copy(ones_hbm, ones_v)
    plsc.subcore_barrier()
    w = s * NC + c
    nchunks = EP // (NC * NS * CHUNK)
    def chunk(k, carry):
        rb = (w * nchunks + k) * IR
        pltpu.sync_copy(dst2d.at[pl.ds(rb, IR)], idx_v)
        for j in range(IR):
            pltpu.sync_copy(ones_v, acc.at[idx_v.at[j]], add=True)
        return carry
    lax.fori_loop(0, nchunks, chunk, 0)
    plsc.subcore_barrier()
    @pl.when(c == 0)
    def _():
        pltpu.sync_copy(acc.at[_stripe(s)], out_a.at[_stripe(s)])
    @pl.when(c == 1)
    def _():
        pltpu.sync_copy(acc.at[_stripe(s)], out_b.at[_stripe(s)])


def _agg_body(edge_split, src_a2d, src_b2d, dst2d, table8, zeros_hbm,
              out, src_v, dst_v, rows_v, acc, sem_g, sem_s):
    c = lax.axis_index("c")
    s = lax.axis_index("s")
    pltpu.sync_copy(zeros_hbm.at[_stripe(s)], acc.at[_stripe(s)])
    plsc.subcore_barrier()
    if edge_split:
        w = s * NC + c
        nchunks = EP // (NC * NS * CHUNK)
        _edge_loop(src_a2d, dst2d, table8, acc, src_v, dst_v, rows_v,
                   sem_g, sem_s, w * nchunks * IR, nchunks)
    else:
        nchunks = EP // (NS * CHUNK)
        row_start = s * nchunks * IR
        @pl.when(c == 0)
        def _():
            _edge_loop(src_a2d, dst2d, table8, acc, src_v, dst_v, rows_v,
                       sem_g, sem_s, row_start, nchunks)
        @pl.when(c == 1)
        def _():
            _edge_loop(src_b2d, dst2d, table8, acc, src_v, dst_v, rows_v,
                       sem_g, sem_s, row_start, nchunks)
    plsc.subcore_barrier()
    @pl.when(c == 0)
    def _():
        pltpu.sync_copy(acc.at[_stripe(s)],
                        out.at[_stripe(s), pl.ds(0, 16)])
    @pl.when(c == 1)
    def _():
        pltpu.sync_copy(acc.at[_stripe(s)],
                        out.at[_stripe(s), pl.ds(16, 16)])


@functools.cache
def _sc_kernels():
    params = pltpu.CompilerParams(use_tc_tiling_on_sc=False)
    deg = pl.kernel(
        _deg_body,
        out_type=[jax.ShapeDtypeStruct((NP,), jnp.float32)] * 2,
        mesh=_sc_mesh(),
        compiler_params=params,
        scratch_types=[
            pltpu.VMEM((IR, SUB), jnp.int32),
            pltpu.VMEM((SUB,), jnp.float32),
            pltpu.VMEM_SHARED((NP,), jnp.float32),
        ],
    )
    aggs = []
    for edge_split in (True, False):
        aggs.append(pl.kernel(
            functools.partial(_agg_body, edge_split),
            out_type=jax.ShapeDtypeStruct((NP, 128), jnp.float32),
            mesh=_sc_mesh(),
            compiler_params=params,
            scratch_types=[
                pltpu.VMEM((IR, SUB), jnp.int32),
                pltpu.VMEM((IR, SUB), jnp.int32),
                pltpu.VMEM((IR, SUB, 16), jnp.float32),
                pltpu.VMEM_SHARED((NP, 16), jnp.float32),
                pltpu.SemaphoreType.DMA,
                pltpu.SemaphoreType.DMA,
            ],
        ))
    return deg, aggs[0], aggs[1]


def _prep_body(deg_a, deg_b, x_ref, misc_ref):
    deg = deg_a[...] + deg_b[...] + 1.0
    dis = lax.rsqrt(deg)
    g1 = x_ref[...] * dis
    pad = jnp.zeros((BT, 30), jnp.float32)
    tail = jnp.zeros((BT, 95), jnp.float32)
    misc_ref[...] = jnp.concatenate([g1, pad, dis, tail], axis=1)


def _mid1_body(e1, misc, w1, b1, out_ref):
    mv = misc[...]
    ev = e1[...]
    dis = mv[:, 32:33]
    z = (ev[:, 0:2] + ev[:, 16:18] + mv[:, 0:2]) * dis
    h1 = jnp.maximum(jnp.dot(z, w1[...], preferred_element_type=jnp.float32)
                     + b1[...], 0.0)
    g2 = h1 * dis
    tail = jnp.zeros((BT, 95), jnp.float32)
    out_ref[...] = jnp.concatenate([g2, dis, tail], axis=1)


def _mid2_body(e2, g2t, w2, b2, w3, out_ref):
    gv = g2t[...]
    dis = gv[:, 32:33]
    u = (e2[...][:, 0:32] + gv[:, 0:32]) * dis
    h2 = jnp.maximum(jnp.dot(u, w2[...], preferred_element_type=jnp.float32)
                     + b2[...], 0.0)
    t3 = jnp.dot(h2, w3[...], preferred_element_type=jnp.float32)
    g3 = t3 * dis
    tail = jnp.zeros((BT, 95), jnp.float32)
    out_ref[...] = jnp.concatenate([g3, dis, tail], axis=1)


def _final_body(e3, g3t, batch, b3, fcw, fcb, out_ref, acc):
    i = pl.program_id(0)
    @pl.when(i == 0)
    def _():
        acc[...] = jnp.zeros_like(acc)
    gv = g3t[...]
    z = (e3[...][:, 0:32] + gv[:, 0:32]) * gv[:, 32:33]
    h3 = jnp.maximum(z + b3[...], 0.0)
    y = jnp.dot(h3, fcw[...], preferred_element_type=jnp.float32)
    yo = jnp.concatenate([y, jnp.ones_like(y)], axis=1)
    onehot_t = (lax.broadcasted_iota(jnp.int32, (GG, BT), 0)
                == batch[...]).astype(jnp.float32)
    acc[...] += jnp.dot(onehot_t, yo, preferred_element_type=jnp.float32)
    @pl.when(i == NSTEP - 1)
    def _():
        a = acc[...]
        out_ref[...] = a[:, 0:1] / jnp.maximum(a[:, 1:2], 1.0) + fcb[...]


def _row_spec(width):
    return pl.BlockSpec((BT, width), lambda i: (i, 0))


def _full_spec(shape):
    return pl.BlockSpec(shape, lambda i: tuple(0 for _ in shape))


_prep_call = pl.pallas_call(
    _prep_body,
    grid=(NSTEP,),
    in_specs=[_row_spec(1), _row_spec(1), _row_spec(2)],
    out_specs=_row_spec(128),
    out_shape=jax.ShapeDtypeStruct((NP, 128), jnp.float32),
)

_mid1_call = pl.pallas_call(
    _mid1_body,
    grid=(NSTEP,),
    in_specs=[_row_spec(128), _row_spec(128),
              _full_spec((2, 32)), _full_spec((1, 32))],
    out_specs=_row_spec(128),
    out_shape=jax.ShapeDtypeStruct((NP, 128), jnp.float32),
)

_mid2_call = pl.pallas_call(
    _mid2_body,
    grid=(NSTEP,),
    in_specs=[_row_spec(128), _row_spec(128),
              _full_spec((32, 64)), _full_spec((1, 64)),
              _full_spec((64, 32))],
    out_specs=_row_spec(128),
    out_shape=jax.ShapeDtypeStruct((NP, 128), jnp.float32),
)

_final_call = pl.pallas_call(
    _final_body,
    grid=(NSTEP,),
    in_specs=[_row_spec(128), _row_spec(128),
              pl.BlockSpec((1, BT), lambda i: (0, i)),
              _full_spec((1, 32)), _full_spec((32, 1)), _full_spec((1, 1))],
    out_specs=_full_spec((GG, 1)),
    out_shape=jax.ShapeDtypeStruct((GG, 1), jnp.float32),
    scratch_shapes=[pltpu.VMEM((GG, 2), jnp.float32)],
)


def kernel(x, edge_index, batch, W1, b1, W2, b2, W3, b3, fc_W, fc_b):
    f32 = jnp.float32
    i32 = jnp.int32
    npad = EP - EE
    pad_src = (jnp.arange(npad, dtype=i32) * 7919) % NN
    pad_dst = NN + (jnp.arange(npad, dtype=i32) % (NP - NN))
    src = jnp.concatenate([edge_index[0], pad_src])
    src_a2d = (src * 8).reshape(EP // SUB, SUB)
    src_b2d = (src * 8 + 1).reshape(EP // SUB, SUB)
    dst2d = jnp.concatenate([edge_index[1], pad_dst]).reshape(EP // SUB, SUB)

    xp = jnp.zeros((NP, 2), f32).at[:NN].set(x.astype(f32))
    batchp = jnp.full((NP,), GG + 7, i32).at[:NN].set(batch).reshape(1, NP)

    ones_sub = jnp.ones((SUB,), f32)
    z1 = jnp.zeros((NP,), f32)
    z16 = jnp.zeros((NP, 16), f32)

    _deg_call, _agg_es_call, _agg_fs_call = _sc_kernels()
    deg_a, deg_b = _deg_call(dst2d, ones_sub, z1)
    misc1 = _prep_call(deg_a.reshape(NP, 1), deg_b.reshape(NP, 1), xp)
    e1 = _agg_es_call(src_a2d, src_b2d, dst2d,
                      misc1.reshape(NP * 8, 16), z16)
    g2t = _mid1_call(e1, misc1, W1, b1.reshape(1, 32))
    e2 = _agg_fs_call(src_a2d, src_b2d, dst2d,
                      g2t.reshape(NP * 8, 16), z16)
    g3t = _mid2_call(e2, g2t, W2, b2.reshape(1, 64), W3)
    e3 = _agg_fs_call(src_a2d, src_b2d, dst2d,
                      g3t.reshape(NP * 8, 16), z16)
    out = _final_call(e3, g3t, batchp, b3.reshape(1, 32),
                      fc_W, fc_b.reshape(1, 1))
    return out
